# baseline (speedup 1.0000x reference)
import functools

import jax
import jax.numpy as jnp
from jax import lax
from jax.experimental import pallas as pl
from jax.experimental.pallas import tpu as pltpu
from jax.experimental.pallas import tpu_sc as plsc

B = 4096
L = 200
C = 64
V = 1000000
LT = L // 8
NCH = L // 2

_info = plsc.get_sparse_core_info()
NC = _info.num_cores
NS = _info.num_subcores
NW = NC * NS
NBB = B // NW
CW = 2 * NBB
PITCH = 4 * NBB + 1


def _body(xh_hbm, w2_hbm, out_hbm, idx_v, rows_v, outc_v, gsem, osem):
    wid = lax.axis_index("s") * NC + lax.axis_index("c")

    iota = lax.iota(jnp.int32, 16)
    cidx = [iota + cb * 16 for cb in range(4)]

    def start_gather(slot, ch):
        pltpu.async_copy(w2_hbm.at[idx_v.at[ch]], rows_v.at[slot],
                         gsem.at[slot])

    def wait_gather(slot):
        pltpu.make_async_copy(w2_hbm.at[idx_v.at[0]], rows_v.at[slot],
                              gsem.at[slot]).wait()

    def start_out(o, t, q):
        pltpu.async_copy(outc_v.at[o, :, pl.ds(0, CW)],
                         out_hbm.at[:, t, wid, pl.ds(CW * q, CW)],
                         osem.at[o])

    def wait_out(o):
        pltpu.make_async_copy(outc_v.at[o, :, pl.ds(0, CW)],
                              out_hbm.at[:, 0, 0, pl.ds(0, CW)],
                              osem.at[o]).wait()

    def transpose_chunk(slot, o, ch):
        @plsc.parallel_loop(0, CW, 1, unroll=4)
        def _(r):
            jv = jnp.full((16,), r, dtype=jnp.int32)
            for cb in range(4):
                v = rows_v[slot, r, pl.ds(cb * 16, 16)]
                plsc.store_scatter(outc_v.at[o], [cidx[cb], jv], v)

    pltpu.sync_copy(xh_hbm.at[:, wid], idx_v)
    start_gather(0, 0)
    start_gather(1, 1)

    def per_tile_row(t, carry):
        ch = 4 * t
        for q in range(4):
            s = q % 2
            o = q % 2
            wait_gather(s)
            if q >= 2:
                wait_out(o)
            else:
                @pl.when(t > 0)
                def _():
                    wait_out(o)
            transpose_chunk(s, o, ch + q)
            if q < 2:
                @pl.when(ch + q + 2 < NCH)
                def _():
                    start_gather(s, ch + q + 2)
            else:
                @pl.when(t < LT - 1)
                def _():
                    start_gather(s, ch + q + 2)
            start_out(o, t, q)
        return carry

    lax.fori_loop(0, LT, per_tile_row, 0)
    wait_out(0)
    wait_out(1)


@functools.partial(jax.jit, static_argnames=())
def _sc_embed(xh, w2):
    mesh = plsc.VectorSubcoreMesh(core_axis_name="c", subcore_axis_name="s")
    f = pl.kernel(
        _body,
        mesh=mesh,
        out_type=jax.ShapeDtypeStruct((C, LT, NW, 8 * NBB), jnp.float32),
        scratch_types=[
            pltpu.VMEM((NCH, CW), jnp.int32),
            pltpu.VMEM((2, CW, 2 * C), jnp.float32),
            pltpu.VMEM((2, C, CW + 1), jnp.float32),
            pltpu.SemaphoreType.DMA((2,)),
            pltpu.SemaphoreType.DMA((2,)),
        ],
        compiler_params=pltpu.CompilerParams(
            needs_layout_passes=False, use_tc_tiling_on_sc=False),
    )
    return f(xh, w2)


def kernel(x, W):
    xt = (jnp.transpose(x).astype(jnp.int32)
          .reshape(NCH, 2, NW, NBB).transpose(0, 2, 1, 3)
          .reshape(NCH, NW, CW))
    w2 = jnp.pad(W, ((0, 0), (0, C)))
    res = _sc_embed(xt, w2).reshape(C, LT, NW, 8, NBB)
    return res.transpose(2, 4, 0, 1, 3).reshape(B, C, L)


# speedup vs baseline: 1.0075x; 1.0075x over previous
import functools

import jax
import jax.numpy as jnp
from jax import lax
from jax.experimental import pallas as pl
from jax.experimental.pallas import tpu as pltpu
from jax.experimental.pallas import tpu_sc as plsc

B = 4096
L = 200
C = 64
LT = L // 8
NCH = L // 2

_info = plsc.get_sparse_core_info()
NC = _info.num_cores
NS = _info.num_subcores
NW = NC * NS
NBB = B // NW


def _body(xt_hbm, w_hbm, out_hbm, idx_v, rows_v, outc_v, gsem, osem):
    wid = lax.axis_index("s") * NC + lax.axis_index("c")

    iota = lax.iota(jnp.int32, 16)
    cidx = [iota + cb * 16 for cb in range(4)]

    def start_gather(slot, ch):
        pltpu.async_copy(w_hbm.at[idx_v.at[ch]], rows_v.at[slot],
                         gsem.at[slot])

    def wait_gather(slot):
        pltpu.make_async_copy(w_hbm.at[idx_v.at[0]], rows_v.at[slot],
                              gsem.at[slot]).wait()

    def start_out(o, t):
        pltpu.async_copy(outc_v.at[o, :, pl.ds(0, 4 * NBB)],
                         out_hbm.at[:, t, wid, pl.ds(4 * NBB * o, 4 * NBB)],
                         osem.at[o])

    def wait_out(o):
        pltpu.make_async_copy(outc_v.at[o, :, pl.ds(0, 4 * NBB)],
                              out_hbm.at[:, 0, 0, pl.ds(0, 4 * NBB)],
                              osem.at[o]).wait()

    def transpose_chunk(slot, o, qls):
        @plsc.parallel_loop(0, 2 * NBB, 1, unroll=4)
        def _(r):
            jv = jnp.full((16,), qls * NBB + r, dtype=jnp.int32)
            for cb in range(4):
                v = rows_v[slot, r, pl.ds(cb * 16, 16)]
                plsc.store_scatter(outc_v.at[o], [cidx[cb], jv], v)

    pltpu.sync_copy(xt_hbm.at[:, wid], idx_v)
    start_gather(0, 0)
    start_gather(1, 1)

    def per_tile_row(t, carry):
        ch = 4 * t
        for q in range(4):
            s = q % 2
            o = q // 2
            wait_gather(s)
            if q % 2 == 0:
                @pl.when(t > 0)
                def _():
                    wait_out(o)
            transpose_chunk(s, o, 2 * (q % 2))
            if q < 2:
                @pl.when(ch + q + 2 < NCH)
                def _():
                    start_gather(s, ch + q + 2)
            else:
                @pl.when(t < LT - 1)
                def _():
                    start_gather(s, ch + q + 2)
            if q % 2 == 1:
                start_out(o, t)
        return carry

    lax.fori_loop(0, LT, per_tile_row, 0)
    wait_out(0)
    wait_out(1)


@functools.partial(jax.jit, static_argnames=())
def _sc_embed(xt, w):
    mesh = plsc.VectorSubcoreMesh(core_axis_name="c", subcore_axis_name="s")
    f = pl.kernel(
        _body,
        mesh=mesh,
        out_type=jax.ShapeDtypeStruct((C, LT, NW, 8 * NBB), jnp.float32),
        scratch_types=[
            pltpu.VMEM((NCH, 2 * NBB), jnp.int32),
            pltpu.VMEM((2, 2 * NBB, C), jnp.float32),
            pltpu.VMEM((2, C, 4 * NBB + 1), jnp.float32),
            pltpu.SemaphoreType.DMA((2,)),
            pltpu.SemaphoreType.DMA((2,)),
        ],
        compiler_params=pltpu.CompilerParams(
            needs_layout_passes=False, use_tc_tiling_on_sc=False),
    )
    return f(xt, w)


def kernel(x, W):
    xt = (jnp.transpose(x).astype(jnp.int32)
          .reshape(NCH, 2, NW, NBB).transpose(0, 2, 1, 3)
          .reshape(NCH, NW, 2 * NBB))
    res = _sc_embed(xt, W).reshape(C, LT, NW, 8, NBB)
    return res.transpose(2, 4, 0, 1, 3).reshape(B, C, L)
